# baseline (speedup 1.0000x reference)
import jax
import jax.numpy as jnp
from jax.experimental import pallas as pl
from jax.experimental.pallas import tpu as pltpu

N = 10000
NFEAT = 128
NHID = 128
NH = N // 2

BM = 200
BK = 200
NB = N // BM
NBK = NH // BK


def _mega_body(adj_ref, x_ref, w1_ref, b1_ref, w2_ref,
               l1w_ref, l1b_ref, l2w_ref, b2_ref, l2b_ref,
               o_ref, s1_ref, v_ref, z_ref):
    i = pl.program_id(0)

    @pl.when(i == 0)
    def _():
        s1_ref[...] = jnp.dot(x_ref[...], w1_ref[...],
                              preferred_element_type=jnp.float32)

    @pl.when(i < NB)
    def _():
        h = jnp.dot(adj_ref[...], s1_ref[...],
                    preferred_element_type=jnp.float32)
        h = jnp.maximum(h + b1_ref[...], 0.0)
        v_ref[pl.ds(i * BM, BM), :] = jnp.dot(
            h, w2_ref[...], preferred_element_type=jnp.float32)

    @pl.when((i >= NB) & (i < 2 * NB))
    def _():
        j = i - NB
        z_ref[pl.ds(j * BM, BM), :] = (
            jnp.dot(adj_ref[...], v_ref[...],
                    preferred_element_type=jnp.float32) + b2_ref[...])

    @pl.when(i >= 2 * NB)
    def _():
        k = i - 2 * NB
        h = jnp.dot(l1w_ref[...], z_ref[...],
                    preferred_element_type=jnp.float32)
        h = jnp.maximum(h + l1b_ref[...], 0.0)
        part = jnp.sum(h * l2w_ref[...]).reshape(1, 1)

        @pl.when(k == 0)
        def _():
            o_ref[...] = part + l2b_ref[...]

        @pl.when(k > 0)
        def _():
            o_ref[...] += part


def _adj_row(i):
    return (jnp.where(i < NB, i, jnp.where(i < 2 * NB, i - NB, NB - 1)), 0)


def _l1_row(i):
    return (jnp.clip(i - 2 * NB, 0, NBK - 1), 0)


def kernel(x, adj, W1, b1, W2, b2, L1_W, L1_b, L2_W, L2_b):
    x2 = x[0]
    adj2 = adj[0]
    b1r = b1.reshape(1, NHID)
    b2r = b2.reshape(1, 1)
    l1b = L1_b.reshape(NH, 1)
    l2w = L2_W.reshape(NH, 1)
    l2b = L2_b.reshape(1, 1)

    out = pl.pallas_call(
        _mega_body,
        grid=(2 * NB + NBK,),
        in_specs=[
            pl.BlockSpec((BM, N), _adj_row),
            pl.BlockSpec((N, NFEAT), lambda i: (0, 0)),
            pl.BlockSpec((NFEAT, NHID), lambda i: (0, 0)),
            pl.BlockSpec((1, NHID), lambda i: (0, 0)),
            pl.BlockSpec((NHID, 1), lambda i: (0, 0)),
            pl.BlockSpec((BK, N), _l1_row),
            pl.BlockSpec((BK, 1), _l1_row),
            pl.BlockSpec((BK, 1), _l1_row),
            pl.BlockSpec((1, 1), lambda i: (0, 0)),
            pl.BlockSpec((1, 1), lambda i: (0, 0)),
        ],
        out_specs=pl.BlockSpec((1, 1), lambda i: (0, 0)),
        out_shape=jax.ShapeDtypeStruct((1, 1), jnp.float32),
        scratch_shapes=[
            pltpu.VMEM((N, NHID), jnp.float32),
            pltpu.VMEM((N, 1), jnp.float32),
            pltpu.VMEM((N, 1), jnp.float32),
        ],
    )(adj2, x2, W1, b1r, W2, L1_W, l1b, l2w, b2r, l2b)

    return out
